# Initial kernel scaffold; baseline (speedup 1.0000x reference)
import jax
import jax.numpy as jnp
from jax.experimental import pallas as pl


def kernel(x, wq, bq, wk, bk, wv, bv, wr, br, gamma):
    raise NotImplementedError("write your pallas kernel here")


# speedup vs baseline: 1.4351x; 1.4351x over previous
import functools

import jax
import jax.numpy as jnp
from jax.experimental import pallas as pl
from jax.experimental.pallas import tpu as pltpu


def _swa_kernel(
    x_ref,
    wq_ref,
    bq_ref,
    wk_ref,
    bk_ref,
    wv_ref,
    bv_ref,
    wr_ref,
    br_ref,
    gamma_ref,
    seg_ref,
    feat_ref,
):
    WB, ws, _, C = x_ref.shape
    n_tok = ws * ws
    rows = WB * n_tok
    xf = x_ref[...].reshape(rows, C)
    cdt = wq_ref.dtype
    xc = xf.astype(cdt)
    K = wq_ref.shape[1]
    f32 = jnp.float32

    q = jnp.dot(xc, wq_ref[...], preferred_element_type=f32) + bq_ref[...]
    k = jnp.dot(xc, wk_ref[...], preferred_element_type=f32) + bk_ref[...]
    v = jnp.dot(xc, wv_ref[...], preferred_element_type=f32) + bv_ref[...]

    seg_ref[...] = q.reshape(WB, ws, ws, K)

    q3 = q.astype(cdt).reshape(WB, n_tok, K)
    k3 = k.astype(cdt).reshape(WB, n_tok, K)
    v3 = v.astype(cdt).reshape(WB, n_tok, C)

    s = jnp.einsum("bnk,bmk->bnm", q3, k3, preferred_element_type=f32)
    s = s - jnp.max(s, axis=-1, keepdims=True)
    p = jnp.exp(s)
    p = p * pl.reciprocal(jnp.sum(p, axis=-1, keepdims=True), approx=True)

    o = jnp.einsum("bnm,bmc->bnc", p.astype(cdt), v3, preferred_element_type=f32)

    o2 = o.reshape(rows, C).astype(cdt)
    r = jnp.dot(o2, wr_ref[...], preferred_element_type=f32) + br_ref[...]
    feat_ref[...] = ((gamma_ref[0] * r + xf)).reshape(WB, ws, ws, C)


def kernel(x, wq, bq, wk, bk, wv, bv, wr, br, gamma):
    B, H, W, C = x.shape
    ws = 8
    K = wq.shape[1]

    WB = 4
    while B % WB:
        WB //= 2
    grid = (B // WB, H // ws, W // ws)

    cdt = jnp.bfloat16
    wqc, wkc, wvc, wrc = (w.astype(cdt) for w in (wq, wk, wv, wr))

    wmap = lambda b, i, j: (0, 0)
    xmap = lambda b, i, j: (b, i, j, 0)

    seg, feat = pl.pallas_call(
        _swa_kernel,
        out_shape=(
            jax.ShapeDtypeStruct((B, H, W, K), jnp.float32),
            jax.ShapeDtypeStruct((B, H, W, C), jnp.float32),
        ),
        grid=grid,
        in_specs=[
            pl.BlockSpec((WB, ws, ws, C), xmap),
            pl.BlockSpec((C, K), wmap),
            pl.BlockSpec((1, K), wmap),
            pl.BlockSpec((C, K), wmap),
            pl.BlockSpec((1, K), wmap),
            pl.BlockSpec((C, C), wmap),
            pl.BlockSpec((1, C), wmap),
            pl.BlockSpec((C, C), wmap),
            pl.BlockSpec((1, C), wmap),
            pl.BlockSpec(memory_space=pltpu.MemorySpace.SMEM),
        ],
        out_specs=[
            pl.BlockSpec((WB, ws, ws, K), xmap),
            pl.BlockSpec((WB, ws, ws, C), xmap),
        ],
        compiler_params=pltpu.CompilerParams(
            dimension_semantics=("parallel", "parallel", "parallel")),
    )(x, wqc, bq, wkc, bk, wvc, bv, wrc, br, gamma)

    return seg, feat


# speedup vs baseline: 3.2172x; 2.2419x over previous
import functools

import jax
import jax.numpy as jnp
from jax.experimental import pallas as pl
from jax.experimental.pallas import tpu as pltpu


def _swa_kernel(
    x_ref,
    wq_ref,
    bq_ref,
    wk_ref,
    bk_ref,
    wv_ref,
    bv_ref,
    wr_ref,
    br_ref,
    gamma_ref,
    seg_ref,
    feat_ref,
):
    WB, ws, _, C = x_ref.shape
    n_tok = ws * ws
    rows = WB * n_tok
    xf = x_ref[...].reshape(rows, C)
    cdt = wq_ref.dtype
    xc = xf.astype(cdt)
    K = wq_ref.shape[1]
    f32 = jnp.float32

    q = jnp.dot(xc, wq_ref[...], preferred_element_type=f32) + bq_ref[...]
    k = jnp.dot(xc, wk_ref[...], preferred_element_type=f32) + bk_ref[...]
    v = jnp.dot(xc, wv_ref[...], preferred_element_type=f32) + bv_ref[...]

    seg_ref[...] = q.reshape(WB, ws, ws, K)

    q3 = q.astype(cdt).reshape(WB, n_tok, K)
    k3 = k.astype(cdt).reshape(WB, n_tok, K)
    v3 = v.astype(cdt).reshape(WB, n_tok, C)

    s = jnp.einsum("bnk,bmk->bnm", q3, k3, preferred_element_type=f32)
    s = s - jnp.max(s, axis=-1, keepdims=True)
    p = jnp.exp(s)
    p = p * pl.reciprocal(jnp.sum(p, axis=-1, keepdims=True), approx=True)

    o = jnp.einsum("bnm,bmc->bnc", p.astype(cdt), v3, preferred_element_type=f32)

    o2 = o.reshape(rows, C).astype(cdt)
    r = jnp.dot(o2, wr_ref[...], preferred_element_type=f32) + br_ref[...]
    feat_ref[...] = ((gamma_ref[0] * r + xf)).reshape(WB, ws, ws, C)


def kernel(x, wq, bq, wk, bk, wv, bv, wr, br, gamma):
    B, H, W, C = x.shape
    ws = 8
    K = wq.shape[1]

    WB = 16
    while B % WB:
        WB //= 2
    grid = (B // WB, H // ws, W // ws)

    cdt = jnp.bfloat16
    wqc, wkc, wvc, wrc = (w.astype(cdt) for w in (wq, wk, wv, wr))

    wmap = lambda b, i, j: (0, 0)
    xmap = lambda b, i, j: (b, i, j, 0)

    seg, feat = pl.pallas_call(
        _swa_kernel,
        out_shape=(
            jax.ShapeDtypeStruct((B, H, W, K), jnp.float32),
            jax.ShapeDtypeStruct((B, H, W, C), jnp.float32),
        ),
        grid=grid,
        in_specs=[
            pl.BlockSpec((WB, ws, ws, C), xmap),
            pl.BlockSpec((C, K), wmap),
            pl.BlockSpec((1, K), wmap),
            pl.BlockSpec((C, K), wmap),
            pl.BlockSpec((1, K), wmap),
            pl.BlockSpec((C, C), wmap),
            pl.BlockSpec((1, C), wmap),
            pl.BlockSpec((C, C), wmap),
            pl.BlockSpec((1, C), wmap),
            pl.BlockSpec(memory_space=pltpu.MemorySpace.SMEM),
        ],
        out_specs=[
            pl.BlockSpec((WB, ws, ws, K), xmap),
            pl.BlockSpec((WB, ws, ws, C), xmap),
        ],
        compiler_params=pltpu.CompilerParams(
            dimension_semantics=("parallel", "parallel", "parallel")),
    )(x, wqc, bq, wkc, bk, wvc, bv, wrc, br, gamma)

    return seg, feat


# speedup vs baseline: 4.1860x; 1.3011x over previous
import functools

import jax
import jax.numpy as jnp
from jax.experimental import pallas as pl
from jax.experimental.pallas import tpu as pltpu


def _swa_kernel(
    x_ref,
    wq_ref,
    bq_ref,
    wk_ref,
    bk_ref,
    wv_ref,
    bv_ref,
    wr_ref,
    br_ref,
    gamma_ref,
    seg_ref,
    feat_ref,
    *,
    n_tok,
):
    d0, d1, d2, C = x_ref.shape
    rows = d0 * d1 * d2
    n_win = rows // n_tok
    xf = x_ref[...].reshape(rows, C)
    cdt = wq_ref.dtype
    xc = xf.astype(cdt)
    K = wq_ref.shape[1]
    f32 = jnp.float32

    q = jnp.dot(xc, wq_ref[...], preferred_element_type=f32) + bq_ref[...]
    k = jnp.dot(xc, wk_ref[...], preferred_element_type=f32) + bk_ref[...]
    v = jnp.dot(xc, wv_ref[...], preferred_element_type=f32) + bv_ref[...]

    seg_ref[...] = q.reshape(seg_ref.shape)

    q3 = q.astype(cdt).reshape(n_win, n_tok, K)
    k3 = k.astype(cdt).reshape(n_win, n_tok, K)
    v3 = v.astype(cdt).reshape(n_win, n_tok, C)

    s = jnp.einsum("bnk,bmk->bnm", q3, k3, preferred_element_type=f32)
    s = s - jnp.max(s, axis=-1, keepdims=True)
    p = jnp.exp(s)
    p = p * pl.reciprocal(jnp.sum(p, axis=-1, keepdims=True), approx=True)

    o = jnp.einsum("bnm,bmc->bnc", p.astype(cdt), v3, preferred_element_type=f32)

    o2 = o.reshape(rows, C).astype(cdt)
    r = jnp.dot(o2, wr_ref[...], preferred_element_type=f32) + br_ref[...]
    feat_ref[...] = (gamma_ref[0] * r + xf).reshape(feat_ref.shape)


def kernel(x, wq, bq, wk, bk, wv, bv, wr, br, gamma):
    B, H, W, C = x.shape
    ws = 8
    K = wq.shape[1]

    HB = 2 * ws
    while H % HB:
        HB //= 2
    grid = (H // HB, W // ws)

    cdt = jnp.bfloat16
    wqc, wkc, wvc, wrc = (w.astype(cdt) for w in (wq, wk, wv, wr))

    wmap = lambda i, j: (0, 0)
    xmap = lambda i, j: (0, i, j, 0)

    seg, feat = pl.pallas_call(
        functools.partial(_swa_kernel, n_tok=ws * ws),
        out_shape=(
            jax.ShapeDtypeStruct((B, H, W, K), jnp.float32),
            jax.ShapeDtypeStruct((B, H, W, C), jnp.float32),
        ),
        grid=grid,
        in_specs=[
            pl.BlockSpec((B, HB, ws, C), xmap),
            pl.BlockSpec((C, K), wmap),
            pl.BlockSpec((1, K), wmap),
            pl.BlockSpec((C, K), wmap),
            pl.BlockSpec((1, K), wmap),
            pl.BlockSpec((C, C), wmap),
            pl.BlockSpec((1, C), wmap),
            pl.BlockSpec((C, C), wmap),
            pl.BlockSpec((1, C), wmap),
            pl.BlockSpec(memory_space=pltpu.MemorySpace.SMEM),
        ],
        out_specs=[
            pl.BlockSpec((B, HB, ws, K), xmap),
            pl.BlockSpec((B, HB, ws, C), xmap),
        ],
        compiler_params=pltpu.CompilerParams(
            dimension_semantics=("parallel", "parallel")),
    )(x, wqc, bq, wkc, bk, wvc, bv, wrc, br, gamma)

    return seg, feat


# speedup vs baseline: 4.7959x; 1.1457x over previous
import functools

import jax
import jax.numpy as jnp
from jax.experimental import pallas as pl
from jax.experimental.pallas import tpu as pltpu


def _swa_kernel(
    x_ref,
    wq_ref,
    bq_ref,
    wk_ref,
    bk_ref,
    wv_ref,
    bv_ref,
    wr_ref,
    br_ref,
    gamma_ref,
    seg_ref,
    feat_ref,
    *,
    n_tok,
):
    d0, d1, d2, C = x_ref.shape
    rows = d0 * d1 * d2
    n_win = rows // n_tok
    xf = x_ref[...].reshape(rows, C)
    cdt = wq_ref.dtype
    xc = xf.astype(cdt)
    K = wq_ref.shape[1]
    f32 = jnp.float32

    q = jnp.dot(xc, wq_ref[...], preferred_element_type=f32) + bq_ref[...]
    k = jnp.dot(xc, wk_ref[...], preferred_element_type=f32) + bk_ref[...]
    v = jnp.dot(xc, wv_ref[...], preferred_element_type=f32) + bv_ref[...]

    seg_ref[...] = q.reshape(seg_ref.shape)

    q3 = q.astype(cdt).reshape(n_win, n_tok, K)
    k3 = k.astype(cdt).reshape(n_win, n_tok, K)
    v3 = v.astype(cdt).reshape(n_win, n_tok, C)

    s = jnp.einsum("bnk,bmk->bnm", q3, k3, preferred_element_type=f32)
    s = s - jnp.max(s, axis=-1, keepdims=True)
    p = jnp.exp(s)
    p = p * pl.reciprocal(jnp.sum(p, axis=-1, keepdims=True), approx=True)

    o = jnp.einsum("bnm,bmc->bnc", p.astype(cdt), v3, preferred_element_type=f32)

    o2 = o.reshape(rows, C).astype(cdt)
    r = jnp.dot(o2, wr_ref[...], preferred_element_type=f32) + br_ref[...]
    feat_ref[...] = (gamma_ref[0] * r + xf).reshape(feat_ref.shape)


def kernel(x, wq, bq, wk, bk, wv, bv, wr, br, gamma):
    B, H, W, C = x.shape
    ws = 8
    K = wq.shape[1]

    HB = 4 * ws
    while H % HB:
        HB //= 2
    grid = (H // HB, W // ws)

    cdt = jnp.bfloat16
    wqc, wkc, wvc, wrc = (w.astype(cdt) for w in (wq, wk, wv, wr))

    wmap = lambda i, j: (0, 0)
    xmap = lambda i, j: (0, i, j, 0)

    seg, feat = pl.pallas_call(
        functools.partial(_swa_kernel, n_tok=ws * ws),
        out_shape=(
            jax.ShapeDtypeStruct((B, H, W, K), jnp.float32),
            jax.ShapeDtypeStruct((B, H, W, C), jnp.float32),
        ),
        grid=grid,
        in_specs=[
            pl.BlockSpec((B, HB, ws, C), xmap),
            pl.BlockSpec((C, K), wmap),
            pl.BlockSpec((1, K), wmap),
            pl.BlockSpec((C, K), wmap),
            pl.BlockSpec((1, K), wmap),
            pl.BlockSpec((C, C), wmap),
            pl.BlockSpec((1, C), wmap),
            pl.BlockSpec((C, C), wmap),
            pl.BlockSpec((1, C), wmap),
            pl.BlockSpec(memory_space=pltpu.MemorySpace.SMEM),
        ],
        out_specs=[
            pl.BlockSpec((B, HB, ws, K), xmap),
            pl.BlockSpec((B, HB, ws, C), xmap),
        ],
        compiler_params=pltpu.CompilerParams(
            dimension_semantics=("parallel", "parallel")),
    )(x, wqc, bq, wkc, bk, wvc, bv, wrc, br, gamma)

    return seg, feat


# speedup vs baseline: 5.0310x; 1.0490x over previous
import functools

import jax
import jax.numpy as jnp
from jax.experimental import pallas as pl
from jax.experimental.pallas import tpu as pltpu


def _swa_kernel(
    x_ref,
    wq_ref,
    bq_ref,
    wk_ref,
    bk_ref,
    wv_ref,
    bv_ref,
    wr_ref,
    br_ref,
    gamma_ref,
    seg_ref,
    feat_ref,
    *,
    n_tok,
):
    d0, d1, d2, C = x_ref.shape
    rows = d0 * d1 * d2
    n_win = rows // n_tok
    xf = x_ref[...].reshape(rows, C)
    cdt = wq_ref.dtype
    xc = xf.astype(cdt)
    K = wq_ref.shape[1]
    f32 = jnp.float32

    q = jnp.dot(xc, wq_ref[...], preferred_element_type=f32) + bq_ref[...]
    k = jnp.dot(xc, wk_ref[...], preferred_element_type=f32) + bk_ref[...]
    v = jnp.dot(xc, wv_ref[...], preferred_element_type=f32) + bv_ref[...]

    seg_ref[...] = q.reshape(seg_ref.shape)

    q3 = q.astype(cdt).reshape(n_win, n_tok, K)
    k3 = k.astype(cdt).reshape(n_win, n_tok, K)
    v3 = v.astype(cdt).reshape(n_win, n_tok, C)

    s = jnp.einsum("bnk,bmk->bnm", q3, k3, preferred_element_type=f32)
    s = s - jnp.max(s, axis=-1, keepdims=True)
    p = jnp.exp(s)
    p = p * pl.reciprocal(jnp.sum(p, axis=-1, keepdims=True), approx=True)

    o = jnp.einsum("bnm,bmc->bnc", p.astype(cdt), v3, preferred_element_type=f32)

    o2 = o.reshape(rows, C).astype(cdt)
    r = jnp.dot(o2, wr_ref[...], preferred_element_type=f32) + br_ref[...]
    feat_ref[...] = (gamma_ref[0] * r + xf).reshape(feat_ref.shape)


def kernel(x, wq, bq, wk, bk, wv, bv, wr, br, gamma):
    B, H, W, C = x.shape
    ws = 8
    K = wq.shape[1]

    HB = 8 * ws
    while H % HB:
        HB //= 2
    grid = (H // HB, W // ws)

    cdt = jnp.bfloat16
    wqc, wkc, wvc, wrc = (w.astype(cdt) for w in (wq, wk, wv, wr))

    wmap = lambda i, j: (0, 0)
    xmap = lambda i, j: (0, i, j, 0)

    seg, feat = pl.pallas_call(
        functools.partial(_swa_kernel, n_tok=ws * ws),
        out_shape=(
            jax.ShapeDtypeStruct((B, H, W, K), jnp.float32),
            jax.ShapeDtypeStruct((B, H, W, C), jnp.float32),
        ),
        grid=grid,
        in_specs=[
            pl.BlockSpec((B, HB, ws, C), xmap),
            pl.BlockSpec((C, K), wmap),
            pl.BlockSpec((1, K), wmap),
            pl.BlockSpec((C, K), wmap),
            pl.BlockSpec((1, K), wmap),
            pl.BlockSpec((C, C), wmap),
            pl.BlockSpec((1, C), wmap),
            pl.BlockSpec((C, C), wmap),
            pl.BlockSpec((1, C), wmap),
            pl.BlockSpec(memory_space=pltpu.MemorySpace.SMEM),
        ],
        out_specs=[
            pl.BlockSpec((B, HB, ws, K), xmap),
            pl.BlockSpec((B, HB, ws, C), xmap),
        ],
        compiler_params=pltpu.CompilerParams(
            dimension_semantics=("parallel", "parallel")),
    )(x, wqc, bq, wkc, bk, wvc, bv, wrc, br, gamma)

    return seg, feat


# speedup vs baseline: 5.4678x; 1.0868x over previous
import functools

import jax
import jax.numpy as jnp
from jax.experimental import pallas as pl
from jax.experimental.pallas import tpu as pltpu


def _swa_kernel(
    x_ref,
    wq_ref,
    bq_ref,
    wk_ref,
    bk_ref,
    wv_ref,
    bv_ref,
    wr_ref,
    br_ref,
    gamma_ref,
    seg_ref,
    feat_ref,
    *,
    n_tok,
):
    d0, d1, d2, C = x_ref.shape
    rows = d0 * d1 * d2
    n_win = rows // n_tok
    xf = x_ref[...].reshape(rows, C)
    cdt = jnp.bfloat16
    xc = xf.astype(cdt)
    K = wq_ref.shape[1]
    f32 = jnp.float32

    wqc = wq_ref[...].astype(cdt)
    wkc = wk_ref[...].astype(cdt)
    wvc = wv_ref[...].astype(cdt)
    wrc = wr_ref[...].astype(cdt)

    q = jnp.dot(xc, wqc, preferred_element_type=f32) + bq_ref[...]
    k = jnp.dot(xc, wkc, preferred_element_type=f32) + bk_ref[...]
    v = jnp.dot(xc, wvc, preferred_element_type=f32) + bv_ref[...]

    seg_ref[...] = q.reshape(seg_ref.shape)

    q3 = q.astype(cdt).reshape(n_win, n_tok, K)
    k3 = k.astype(cdt).reshape(n_win, n_tok, K)
    v3 = v.astype(cdt).reshape(n_win, n_tok, C)

    s = jnp.einsum("bnk,bmk->bnm", q3, k3, preferred_element_type=f32)
    s = s - jnp.max(s, axis=-1, keepdims=True)
    p = jnp.exp(s)
    p = p * pl.reciprocal(jnp.sum(p, axis=-1, keepdims=True), approx=True)

    o = jnp.einsum("bnm,bmc->bnc", p.astype(cdt), v3, preferred_element_type=f32)

    o2 = o.reshape(rows, C).astype(cdt)
    r = jnp.dot(o2, wrc, preferred_element_type=f32) + br_ref[...]
    feat_ref[...] = (gamma_ref[0] * r + xf).reshape(feat_ref.shape)


def kernel(x, wq, bq, wk, bk, wv, bv, wr, br, gamma):
    B, H, W, C = x.shape
    ws = 8
    K = wq.shape[1]

    HB = 8 * ws
    while H % HB:
        HB //= 2
    grid = (H // HB, W // ws)

    wmap = lambda i, j: (0, 0)
    xmap = lambda i, j: (0, i, j, 0)

    seg, feat = pl.pallas_call(
        functools.partial(_swa_kernel, n_tok=ws * ws),
        out_shape=(
            jax.ShapeDtypeStruct((B, H, W, K), jnp.float32),
            jax.ShapeDtypeStruct((B, H, W, C), jnp.float32),
        ),
        grid=grid,
        in_specs=[
            pl.BlockSpec((B, HB, ws, C), xmap),
            pl.BlockSpec((C, K), wmap),
            pl.BlockSpec((1, K), wmap),
            pl.BlockSpec((C, K), wmap),
            pl.BlockSpec((1, K), wmap),
            pl.BlockSpec((C, C), wmap),
            pl.BlockSpec((1, C), wmap),
            pl.BlockSpec((C, C), wmap),
            pl.BlockSpec((1, C), wmap),
            pl.BlockSpec(memory_space=pltpu.MemorySpace.SMEM),
        ],
        out_specs=[
            pl.BlockSpec((B, HB, ws, K), xmap),
            pl.BlockSpec((B, HB, ws, C), xmap),
        ],
        compiler_params=pltpu.CompilerParams(
            dimension_semantics=("parallel", "parallel")),
    )(x, wq, bq, wk, bk, wv, bv, wr, br, gamma)

    return seg, feat


# speedup vs baseline: 6.1313x; 1.1214x over previous
import functools

import jax
import jax.numpy as jnp
from jax.experimental import pallas as pl
from jax.experimental.pallas import tpu as pltpu


def _swa_kernel(
    x_ref,
    wq_ref,
    bq_ref,
    wk_ref,
    bk_ref,
    wv_ref,
    bv_ref,
    wr_ref,
    br_ref,
    gamma_ref,
    seg_ref,
    feat_ref,
    *,
    n_tok,
):
    d0, d1, d2, C = x_ref.shape
    rows = d0 * d1 * d2
    n_win = rows // n_tok
    xf = x_ref[...].reshape(rows, C)
    cdt = jnp.bfloat16
    xc = xf.astype(cdt)
    K = wq_ref.shape[1]
    f32 = jnp.float32

    wqkv = jnp.concatenate(
        [wq_ref[...].astype(cdt), wk_ref[...].astype(cdt),
         wv_ref[...].astype(cdt)], axis=1)
    wrc = wr_ref[...].astype(cdt)

    qkv = jnp.dot(xc, wqkv, preferred_element_type=f32)
    q = qkv[:, :K] + bq_ref[...]
    k = qkv[:, K:2 * K] + bk_ref[...]
    v = qkv[:, 2 * K:] + bv_ref[...]

    seg_ref[...] = q.reshape(seg_ref.shape)

    q3 = q.astype(cdt).reshape(n_win, n_tok, K)
    k3 = k.astype(cdt).reshape(n_win, n_tok, K)
    v3 = v.astype(cdt).reshape(n_win, n_tok, C)

    s = jnp.einsum("bnk,bmk->bnm", q3, k3, preferred_element_type=f32)
    s = s - jnp.max(s, axis=-1, keepdims=True)
    p = jnp.exp(s)
    p = p * pl.reciprocal(jnp.sum(p, axis=-1, keepdims=True), approx=True)

    o = jnp.einsum("bnm,bmc->bnc", p.astype(cdt), v3, preferred_element_type=f32)

    o2 = o.reshape(rows, C).astype(cdt)
    r = jnp.dot(o2, wrc, preferred_element_type=f32) + br_ref[...]
    feat_ref[...] = (gamma_ref[0] * r + xf).reshape(feat_ref.shape)


def kernel(x, wq, bq, wk, bk, wv, bv, wr, br, gamma):
    B, H, W, C = x.shape
    ws = 8
    K = wq.shape[1]

    HB = 8 * ws
    while H % HB:
        HB //= 2
    grid = (H // HB, W // ws)

    wmap = lambda i, j: (0, 0)
    xmap = lambda i, j: (0, i, j, 0)

    seg, feat = pl.pallas_call(
        functools.partial(_swa_kernel, n_tok=ws * ws),
        out_shape=(
            jax.ShapeDtypeStruct((B, H, W, K), jnp.float32),
            jax.ShapeDtypeStruct((B, H, W, C), jnp.float32),
        ),
        grid=grid,
        in_specs=[
            pl.BlockSpec((B, HB, ws, C), xmap),
            pl.BlockSpec((C, K), wmap),
            pl.BlockSpec((1, K), wmap),
            pl.BlockSpec((C, K), wmap),
            pl.BlockSpec((1, K), wmap),
            pl.BlockSpec((C, C), wmap),
            pl.BlockSpec((1, C), wmap),
            pl.BlockSpec((C, C), wmap),
            pl.BlockSpec((1, C), wmap),
            pl.BlockSpec(memory_space=pltpu.MemorySpace.SMEM),
        ],
        out_specs=[
            pl.BlockSpec((B, HB, ws, K), xmap),
            pl.BlockSpec((B, HB, ws, C), xmap),
        ],
        compiler_params=pltpu.CompilerParams(
            dimension_semantics=("parallel", "parallel")),
    )(x, wq, bq, wk, bk, wv, bv, wr, br, gamma)

    return seg, feat
